# Initial kernel scaffold; baseline (speedup 1.0000x reference)
import jax
import jax.numpy as jnp
from jax.experimental import pallas as pl


def kernel(anchors, deltas, objectness):
    raise NotImplementedError("write your pallas kernel here")


# speedup vs baseline: 13.8527x; 13.8527x over previous
import functools

import jax
import jax.numpy as jnp
import numpy as np
from jax import lax
from jax.experimental import pallas as pl

N_ANCHORS = 20000
NPAD = 20480
ROWS = 160
LANES = 128
PRE = 1000
CAP = 1024
NMS_THRESH = 0.7
IMG_H = 800.0
IMG_W = 800.0
BBOX_XFORM_CLIP = float(np.log(1000.0 / 16.0))
NEG_INF_SCORE = -1e9
HI = jax.lax.Precision.HIGHEST


def _rpn_body(sc_ref, ax1_ref, ay1_ref, ax2_ref, ay2_ref,
              dx_ref, dy_ref, dw_ref, dh_ref, out_ref):
    f32 = jnp.float32
    score = sc_ref[...]
    ax1 = ax1_ref[...]
    ay1 = ay1_ref[...]
    ax2 = ax2_ref[...]
    ay2 = ay2_ref[...]
    dx = dx_ref[...]
    dy = dy_ref[...]
    dw = dw_ref[...]
    dh = dh_ref[...]

    w = ax2 - ax1
    h = ay2 - ay1
    cx = ax1 + 0.5 * w
    cy = ay1 + 0.5 * h
    dwc = jnp.minimum(dw, BBOX_XFORM_CLIP)
    dhc = jnp.minimum(dh, BBOX_XFORM_CLIP)
    pcx = dx * w + cx
    pcy = dy * h + cy
    pw = jnp.exp(dwc) * w
    ph = jnp.exp(dhc) * h
    x1 = jnp.clip(pcx - 0.5 * pw, 0.0, IMG_W)
    y1 = jnp.clip(pcy - 0.5 * ph, 0.0, IMG_H)
    x2 = jnp.clip(pcx + 0.5 * pw, 0.0, IMG_W)
    y2 = jnp.clip(pcy + 0.5 * ph, 0.0, IMG_H)

    int_min = jnp.int32(-2147483648)
    kb = lax.bitcast_convert_type(score, jnp.int32)
    k = jnp.where(kb >= 0, kb, kb ^ jnp.int32(0x7FFFFFFF))

    def bs_body(b, u):
        bit = lax.shift_left(jnp.int32(1), jnp.int32(31) - b)
        uc = u | bit
        cand = uc ^ int_min
        cnt = jnp.sum((k >= cand).astype(jnp.int32))
        return jnp.where(cnt >= PRE, uc, u)

    u_fin = lax.fori_loop(0, 32, bs_body, jnp.int32(0))
    thr = u_fin ^ int_min
    sel = k >= thr
    self_f = sel.astype(f32)

    io0_l = lax.broadcasted_iota(jnp.int32, (LANES, LANES), 0)
    io1_l = lax.broadcasted_iota(jnp.int32, (LANES, LANES), 1)
    u128 = (io0_l < io1_l).astype(f32)
    within = lax.dot_general(self_f, u128, (((1,), (0,)), ((), ())),
                             precision=HI)
    rowtot = jnp.sum(self_f, axis=1, keepdims=True)
    io0_r = lax.broadcasted_iota(jnp.int32, (ROWS, ROWS), 0)
    io1_r = lax.broadcasted_iota(jnp.int32, (ROWS, ROWS), 1)
    l160 = (io1_r < io0_r).astype(f32)
    rowoff = lax.dot_general(l160, rowtot, (((1,), (0,)), ((), ())),
                             precision=HI)
    pos = within + rowoff
    posx = jnp.where(sel, pos, f32(2.0 * CAP))

    idxf = (lax.broadcasted_iota(jnp.int32, (ROWS, LANES), 0) * LANES
            + lax.broadcasted_iota(jnp.int32, (ROWS, LANES), 1)).astype(f32)

    iota_p = lax.broadcasted_iota(jnp.int32, (CAP, CAP), 0).astype(f32)
    iota_e1 = lax.broadcasted_iota(jnp.int32, (CAP, CAP), 1).astype(f32)

    score_pay = jnp.where(sel, score, 0.0)
    payload = (x1, y1, x2, y2, score_pay, idxf,
               jnp.ones((ROWS, LANES), f32), jnp.zeros((ROWS, LANES), f32))

    def row1024(arr, b):
        ch = arr[8 * b:8 * b + 8, :]
        return jnp.concatenate([ch[s:s + 1, :] for s in range(8)], axis=1)

    c_mat = jnp.zeros((8, CAP), f32)
    for b in range(NPAD // CAP):
        pr = row1024(posx, b)
        ob = (jnp.broadcast_to(pr, (CAP, CAP)) == iota_p).astype(f32)
        vb = jnp.concatenate([row1024(q, b) for q in payload], axis=0)
        c_mat = c_mat + lax.dot_general(vb, ob, (((1,), (1,)), ((), ())),
                                        precision=HI)

    eye = (iota_p == iota_e1).astype(f32)

    def to_col(row):
        return jnp.sum(jnp.broadcast_to(row, (CAP, CAP)) * eye,
                       axis=1, keepdims=True)

    s_row = c_mat[4:5, :]
    i_row = c_mat[5:6, :]
    v_row = c_mat[6:7, :]
    s_col = to_col(s_row)
    i_col = to_col(i_row)
    v_col = to_col(v_row)
    before = ((v_col > v_row)
              | ((v_col == v_row)
                 & ((s_col > s_row)
                    | ((s_col == s_row)
                       & ((i_col < i_row)
                          | ((i_col == i_row) & (iota_p < iota_e1)))))))
    rank_row = jnp.sum(before.astype(f32), axis=0, keepdims=True)
    g_mat = (jnp.broadcast_to(rank_row, (CAP, CAP)) == iota_p).astype(f32)
    s_sorted = lax.dot_general(c_mat, g_mat, (((1,), (1,)), ((), ())),
                               precision=HI)

    bx1r = s_sorted[0:1, :]
    by1r = s_sorted[1:2, :]
    bx2r = s_sorted[2:3, :]
    by2r = s_sorted[3:4, :]
    sc_r = s_sorted[4:5, :]
    bx1c = to_col(bx1r)
    by1c = to_col(by1r)
    bx2c = to_col(bx2r)
    by2c = to_col(by2r)
    area_r = (bx2r - bx1r) * (by2r - by1r)
    area_c = (bx2c - bx1c) * (by2c - by1c)
    wx = jnp.maximum(jnp.minimum(bx2c, bx2r) - jnp.maximum(bx1c, bx1r), 0.0)
    wy = jnp.maximum(jnp.minimum(by2c, by2r) - jnp.maximum(by1c, by1r), 0.0)
    inter = wx * wy
    iou = inter / (area_c + area_r - inter + 1e-9)
    act_pair = (iota_p < f32(PRE)) & (iota_e1 < f32(PRE))
    sup_u = ((iou > NMS_THRESH) & (iota_p < iota_e1) & act_pair).astype(f32)

    iota_row = lax.broadcasted_iota(jnp.int32, (1, CAP), 1).astype(f32)
    act_row = (iota_row < f32(PRE)).astype(f32)

    def nms_cond(carry):
        _, done = carry
        return jnp.logical_not(done)

    def nms_body(carry):
        keep, _ = carry
        cnt = lax.dot_general(keep, sup_u, (((1,), (0,)), ((), ())),
                              precision=HI)
        newk = act_row * (cnt == 0.0).astype(f32)
        done = jnp.all(newk == keep)
        return (newk, done)

    keep, _ = lax.while_loop(nms_cond, nms_body,
                             (act_row, jnp.bool_(False)))

    u_cap = (iota_p < iota_e1).astype(f32)
    pk = lax.dot_general(keep, u_cap, (((1,), (0,)), ((), ())), precision=HI)
    nk = jnp.sum(keep)
    notk = act_row * (1.0 - keep)
    pn = lax.dot_general(notk, u_cap, (((1,), (0,)), ((), ())), precision=HI)
    fpos = jnp.where(keep == 1.0, pk, nk + pn)
    fposx = jnp.where(act_row == 1.0, fpos, f32(2.0 * CAP))
    h_mat = (jnp.broadcast_to(fposx, (CAP, CAP)) == iota_p).astype(f32)
    sc_fin = jnp.where(keep == 1.0, sc_r, f32(NEG_INF_SCORE))
    s_out = jnp.concatenate(
        [bx1r, by1r, bx2r, by2r, sc_fin,
         jnp.zeros((3, CAP), f32)], axis=0)
    out_ref[...] = lax.dot_general(s_out, h_mat, (((1,), (1,)), ((), ())),
                                   precision=HI)


@functools.partial(jax.jit, static_argnames=("interpret",))
def _rpn_call(sc2, ax1, ay1, ax2, ay2, dx, dy, dw, dh, interpret=False):
    return pl.pallas_call(
        _rpn_body,
        out_shape=jax.ShapeDtypeStruct((8, CAP), jnp.float32),
        interpret=interpret,
    )(sc2, ax1, ay1, ax2, ay2, dx, dy, dw, dh)


def kernel(anchors, deltas, objectness, interpret=False):
    f32 = jnp.float32
    sc = jnp.full((NPAD,), -jnp.inf, f32).at[:N_ANCHORS].set(objectness)
    sc2 = sc.reshape(ROWS, LANES)

    def cols(m):
        mp = jnp.zeros((NPAD, 4), f32).at[:N_ANCHORS].set(m)
        return [mp[:, j].reshape(ROWS, LANES) for j in range(4)]

    ax1, ay1, ax2, ay2 = cols(anchors)
    dx, dy, dw, dh = cols(deltas)
    res = _rpn_call(sc2, ax1, ay1, ax2, ay2, dx, dy, dw, dh,
                    interpret=interpret)
    return jnp.transpose(res[0:5, :PRE])


# speedup vs baseline: 14.2694x; 1.0301x over previous
import functools

import jax
import jax.numpy as jnp
import numpy as np
from jax import lax
from jax.experimental import pallas as pl

N_ANCHORS = 20000
NPAD = 20480
ROWS = 160
LANES = 128
PRE = 1000
CAP = 1024
NMS_THRESH = 0.7
IMG_H = 800.0
IMG_W = 800.0
BBOX_XFORM_CLIP = float(np.log(1000.0 / 16.0))
NEG_INF_SCORE = -1e9
HI = jax.lax.Precision.HIGHEST
LO = jax.lax.Precision.DEFAULT


def _rpn_body(sc_ref, ax1_ref, ay1_ref, ax2_ref, ay2_ref,
              dx_ref, dy_ref, dw_ref, dh_ref, out_ref):
    f32 = jnp.float32
    score = sc_ref[...]
    ax1 = ax1_ref[...]
    ay1 = ay1_ref[...]
    ax2 = ax2_ref[...]
    ay2 = ay2_ref[...]
    dx = dx_ref[...]
    dy = dy_ref[...]
    dw = dw_ref[...]
    dh = dh_ref[...]

    w = ax2 - ax1
    h = ay2 - ay1
    cx = ax1 + 0.5 * w
    cy = ay1 + 0.5 * h
    dwc = jnp.minimum(dw, BBOX_XFORM_CLIP)
    dhc = jnp.minimum(dh, BBOX_XFORM_CLIP)
    pcx = dx * w + cx
    pcy = dy * h + cy
    pw = jnp.exp(dwc) * w
    ph = jnp.exp(dhc) * h
    x1 = jnp.clip(pcx - 0.5 * pw, 0.0, IMG_W)
    y1 = jnp.clip(pcy - 0.5 * ph, 0.0, IMG_H)
    x2 = jnp.clip(pcx + 0.5 * pw, 0.0, IMG_W)
    y2 = jnp.clip(pcy + 0.5 * ph, 0.0, IMG_H)

    int_min = jnp.int32(-2147483648)
    kb = lax.bitcast_convert_type(score, jnp.int32)
    k = jnp.where(kb >= 0, kb, kb ^ jnp.int32(0x7FFFFFFF))

    def bs_body(b, u):
        bit = lax.shift_left(jnp.int32(1), jnp.int32(31) - b)
        uc = u | bit
        cand = uc ^ int_min
        cnt = jnp.sum((k >= cand).astype(jnp.int32))
        return jnp.where(cnt >= PRE, uc, u)

    u_fin = lax.fori_loop(0, 32, bs_body, jnp.int32(0))
    thr = u_fin ^ int_min
    sel = k >= thr
    self_f = sel.astype(f32)

    io0_l = lax.broadcasted_iota(jnp.int32, (LANES, LANES), 0)
    io1_l = lax.broadcasted_iota(jnp.int32, (LANES, LANES), 1)
    u128 = (io0_l < io1_l).astype(f32)
    within = lax.dot_general(self_f, u128, (((1,), (0,)), ((), ())),
                             precision=LO)
    rowtot = jnp.sum(self_f, axis=1, keepdims=True)
    io0_r = lax.broadcasted_iota(jnp.int32, (ROWS, ROWS), 0)
    io1_r = lax.broadcasted_iota(jnp.int32, (ROWS, ROWS), 1)
    l160 = (io1_r < io0_r).astype(f32)
    rowoff = lax.dot_general(l160, rowtot, (((1,), (0,)), ((), ())),
                             precision=LO)
    pos = within + rowoff
    posx = jnp.where(sel, pos, f32(2.0 * CAP))

    idxf = (lax.broadcasted_iota(jnp.int32, (ROWS, LANES), 0) * LANES
            + lax.broadcasted_iota(jnp.int32, (ROWS, LANES), 1)).astype(f32)

    iota_p = lax.broadcasted_iota(jnp.int32, (CAP, CAP), 0).astype(f32)
    iota_e1 = lax.broadcasted_iota(jnp.int32, (CAP, CAP), 1).astype(f32)

    score_pay = jnp.where(sel, score, 0.0)
    payload = (x1, y1, x2, y2, score_pay, idxf,
               jnp.ones((ROWS, LANES), f32), jnp.zeros((ROWS, LANES), f32))

    def row1024(arr, b):
        ch = arr[8 * b:8 * b + 8, :]
        return jnp.concatenate([ch[s:s + 1, :] for s in range(8)], axis=1)

    c_mat = jnp.zeros((8, CAP), f32)
    for b in range(NPAD // CAP):
        pr = row1024(posx, b)
        ob = (jnp.broadcast_to(pr, (CAP, CAP)) == iota_p).astype(f32)
        vb = jnp.concatenate([row1024(q, b) for q in payload], axis=0)
        c_mat = c_mat + lax.dot_general(vb, ob, (((1,), (1,)), ((), ())),
                                        precision=HI)

    eye = (iota_p == iota_e1).astype(f32)

    def to_col(row):
        return jnp.sum(jnp.broadcast_to(row, (CAP, CAP)) * eye,
                       axis=1, keepdims=True)

    s_row = c_mat[4:5, :]
    i_row = c_mat[5:6, :]
    v_row = c_mat[6:7, :]
    s_col = to_col(s_row)
    i_col = to_col(i_row)
    v_col = to_col(v_row)
    before = ((v_col > v_row)
              | ((v_col == v_row)
                 & ((s_col > s_row)
                    | ((s_col == s_row)
                       & ((i_col < i_row)
                          | ((i_col == i_row) & (iota_p < iota_e1)))))))
    rank_row = jnp.sum(before.astype(f32), axis=0, keepdims=True)
    g_mat = (jnp.broadcast_to(rank_row, (CAP, CAP)) == iota_p).astype(f32)
    s_sorted = lax.dot_general(c_mat, g_mat, (((1,), (1,)), ((), ())),
                               precision=HI)

    bx1r = s_sorted[0:1, :]
    by1r = s_sorted[1:2, :]
    bx2r = s_sorted[2:3, :]
    by2r = s_sorted[3:4, :]
    sc_r = s_sorted[4:5, :]
    bx1c = to_col(bx1r)
    by1c = to_col(by1r)
    bx2c = to_col(bx2r)
    by2c = to_col(by2r)
    area_r = (bx2r - bx1r) * (by2r - by1r)
    area_c = (bx2c - bx1c) * (by2c - by1c)
    wx = jnp.maximum(jnp.minimum(bx2c, bx2r) - jnp.maximum(bx1c, bx1r), 0.0)
    wy = jnp.maximum(jnp.minimum(by2c, by2r) - jnp.maximum(by1c, by1r), 0.0)
    inter = wx * wy
    iou = inter / (area_c + area_r - inter + 1e-9)
    act_pair = (iota_p < f32(PRE)) & (iota_e1 < f32(PRE))
    sup_u = ((iou > NMS_THRESH) & (iota_p < iota_e1) & act_pair).astype(f32)

    iota_row = lax.broadcasted_iota(jnp.int32, (1, CAP), 1).astype(f32)
    act_row = (iota_row < f32(PRE)).astype(f32)

    def nms_cond(carry):
        _, done = carry
        return jnp.logical_not(done)

    def nms_body(carry):
        keep, _ = carry
        cnt = lax.dot_general(keep, sup_u, (((1,), (0,)), ((), ())),
                              precision=LO)
        newk = act_row * (cnt == 0.0).astype(f32)
        done = jnp.all(newk == keep)
        return (newk, done)

    keep, _ = lax.while_loop(nms_cond, nms_body,
                             (act_row, jnp.bool_(False)))

    u_cap = (iota_p < iota_e1).astype(f32)
    pk = lax.dot_general(keep, u_cap, (((1,), (0,)), ((), ())), precision=LO)
    nk = jnp.sum(keep)
    notk = act_row * (1.0 - keep)
    pn = lax.dot_general(notk, u_cap, (((1,), (0,)), ((), ())), precision=LO)
    fpos = jnp.where(keep == 1.0, pk, nk + pn)
    fposx = jnp.where(act_row == 1.0, fpos, f32(2.0 * CAP))
    h_mat = (jnp.broadcast_to(fposx, (CAP, CAP)) == iota_p).astype(f32)
    sc_fin = jnp.where(keep == 1.0, sc_r, f32(NEG_INF_SCORE))
    s_out = jnp.concatenate(
        [bx1r, by1r, bx2r, by2r, sc_fin,
         jnp.zeros((3, CAP), f32)], axis=0)
    out_ref[...] = lax.dot_general(s_out, h_mat, (((1,), (1,)), ((), ())),
                                   precision=HI)


@functools.partial(jax.jit, static_argnames=("interpret",))
def _rpn_call(sc2, ax1, ay1, ax2, ay2, dx, dy, dw, dh, interpret=False):
    return pl.pallas_call(
        _rpn_body,
        out_shape=jax.ShapeDtypeStruct((8, CAP), jnp.float32),
        interpret=interpret,
    )(sc2, ax1, ay1, ax2, ay2, dx, dy, dw, dh)


def kernel(anchors, deltas, objectness, interpret=False):
    f32 = jnp.float32
    sc = jnp.full((NPAD,), -jnp.inf, f32).at[:N_ANCHORS].set(objectness)
    sc2 = sc.reshape(ROWS, LANES)

    def cols(m):
        mp = jnp.zeros((NPAD, 4), f32).at[:N_ANCHORS].set(m)
        return [mp[:, j].reshape(ROWS, LANES) for j in range(4)]

    ax1, ay1, ax2, ay2 = cols(anchors)
    dx, dy, dw, dh = cols(deltas)
    res = _rpn_call(sc2, ax1, ay1, ax2, ay2, dx, dy, dw, dh,
                    interpret=interpret)
    return jnp.transpose(res[0:5, :PRE])


# speedup vs baseline: 38.2564x; 2.6810x over previous
import functools

import jax
import jax.numpy as jnp
import numpy as np
from jax import lax
from jax.experimental import pallas as pl

N_ANCHORS = 20000
NPAD = 20480
ROWS = 160
LANES = 128
PRE = 1000
CAP = 1024
NMS_THRESH = 0.7
IMG_H = 800.0
IMG_W = 800.0
BBOX_XFORM_CLIP = float(np.log(1000.0 / 16.0))
NEG_INF_SCORE = -1e9
HI = jax.lax.Precision.HIGHEST
LO = jax.lax.Precision.DEFAULT


def _rpn_body(sc_ref, ax1_ref, ay1_ref, ax2_ref, ay2_ref,
              dx_ref, dy_ref, dw_ref, dh_ref, out_ref):
    f32 = jnp.float32
    score = sc_ref[...]
    ax1 = ax1_ref[...]
    ay1 = ay1_ref[...]
    ax2 = ax2_ref[...]
    ay2 = ay2_ref[...]
    dx = dx_ref[...]
    dy = dy_ref[...]
    dw = dw_ref[...]
    dh = dh_ref[...]

    w = ax2 - ax1
    h = ay2 - ay1
    cx = ax1 + 0.5 * w
    cy = ay1 + 0.5 * h
    dwc = jnp.minimum(dw, BBOX_XFORM_CLIP)
    dhc = jnp.minimum(dh, BBOX_XFORM_CLIP)
    pcx = dx * w + cx
    pcy = dy * h + cy
    pw = jnp.exp(dwc) * w
    ph = jnp.exp(dhc) * h
    x1 = jnp.clip(pcx - 0.5 * pw, 0.0, IMG_W)
    y1 = jnp.clip(pcy - 0.5 * ph, 0.0, IMG_H)
    x2 = jnp.clip(pcx + 0.5 * pw, 0.0, IMG_W)
    y2 = jnp.clip(pcy + 0.5 * ph, 0.0, IMG_H)

    int_min = jnp.int32(-2147483648)
    kb = lax.bitcast_convert_type(score, jnp.int32)
    k = jnp.where(kb >= 0, kb, kb ^ jnp.int32(0x7FFFFFFF))

    def bs_body(b, u):
        bit = lax.shift_left(jnp.int32(1), jnp.int32(31) - b)
        uc = u | bit
        cand = uc ^ int_min
        cnt = jnp.sum((k >= cand).astype(jnp.int32))
        return jnp.where(cnt >= PRE, uc, u)

    u_fin = lax.fori_loop(0, 32, bs_body, jnp.int32(0))
    thr = u_fin ^ int_min
    sel = k >= thr
    self_f = sel.astype(f32)

    io0_l = lax.broadcasted_iota(jnp.int32, (LANES, LANES), 0)
    io1_l = lax.broadcasted_iota(jnp.int32, (LANES, LANES), 1)
    u128 = (io0_l < io1_l).astype(f32)
    within = lax.dot_general(self_f, u128, (((1,), (0,)), ((), ())),
                             precision=LO)
    rowtot = jnp.sum(self_f, axis=1, keepdims=True)
    io0_r = lax.broadcasted_iota(jnp.int32, (ROWS, ROWS), 0)
    io1_r = lax.broadcasted_iota(jnp.int32, (ROWS, ROWS), 1)
    l160 = (io1_r < io0_r).astype(f32)
    rowoff = lax.dot_general(l160, rowtot, (((1,), (0,)), ((), ())),
                             precision=LO)
    pos = within + rowoff
    posx = jnp.where(sel, pos, f32(2.0 * CAP))

    idxf = (lax.broadcasted_iota(jnp.int32, (ROWS, LANES), 0) * LANES
            + lax.broadcasted_iota(jnp.int32, (ROWS, LANES), 1)).astype(f32)

    iota_p = lax.broadcasted_iota(jnp.int32, (CAP, CAP), 0).astype(f32)
    iota_e1 = lax.broadcasted_iota(jnp.int32, (CAP, CAP), 1).astype(f32)
    bf16 = jnp.bfloat16

    def split3(x):
        hi = x.astype(bf16)
        r1 = x - hi.astype(f32)
        mid = r1.astype(bf16)
        lo = (r1 - mid.astype(f32)).astype(bf16)
        return jnp.concatenate([hi, mid, lo], axis=0)

    def onehot_matmul(vals, onehot_bf):
        m = vals.shape[0]
        parts = lax.dot_general(split3(vals), onehot_bf,
                                (((1,), (1,)), ((), ())),
                                preferred_element_type=f32)
        return parts[0:m] + parts[m:2 * m] + parts[2 * m:3 * m]

    score_pay = jnp.where(sel, score, 0.0)
    payload = (x1, y1, x2, y2, score_pay, idxf,
               jnp.ones((ROWS, LANES), f32), jnp.zeros((ROWS, LANES), f32))

    def row1024(arr, b):
        ch = arr[8 * b:8 * b + 8, :]
        return jnp.concatenate([ch[s:s + 1, :] for s in range(8)], axis=1)

    c_mat = jnp.zeros((8, CAP), f32)
    for b in range(NPAD // CAP):
        pr = row1024(posx, b)
        ob = (jnp.broadcast_to(pr, (CAP, CAP)) == iota_p).astype(bf16)
        vb = jnp.concatenate([row1024(q, b) for q in payload], axis=0)
        c_mat = c_mat + onehot_matmul(vb, ob)

    eye = (iota_p == iota_e1).astype(f32)

    def to_col(row):
        return jnp.sum(jnp.broadcast_to(row, (CAP, CAP)) * eye,
                       axis=1, keepdims=True)

    s_row = c_mat[4:5, :]
    i_row = c_mat[5:6, :]
    v_row = c_mat[6:7, :]
    s_col = to_col(s_row)
    i_col = to_col(i_row)
    v_col = to_col(v_row)
    before = ((v_col > v_row)
              | ((v_col == v_row)
                 & ((s_col > s_row)
                    | ((s_col == s_row)
                       & ((i_col < i_row)
                          | ((i_col == i_row) & (iota_p < iota_e1)))))))
    rank_row = jnp.sum(before.astype(f32), axis=0, keepdims=True)
    g_mat = (jnp.broadcast_to(rank_row, (CAP, CAP)) == iota_p).astype(bf16)
    s_sorted = onehot_matmul(c_mat, g_mat)

    bx1r = s_sorted[0:1, :]
    by1r = s_sorted[1:2, :]
    bx2r = s_sorted[2:3, :]
    by2r = s_sorted[3:4, :]
    sc_r = s_sorted[4:5, :]
    bx1c = to_col(bx1r)
    by1c = to_col(by1r)
    bx2c = to_col(bx2r)
    by2c = to_col(by2r)
    area_r = (bx2r - bx1r) * (by2r - by1r)
    area_c = (bx2c - bx1c) * (by2c - by1c)
    wx = jnp.maximum(jnp.minimum(bx2c, bx2r) - jnp.maximum(bx1c, bx1r), 0.0)
    wy = jnp.maximum(jnp.minimum(by2c, by2r) - jnp.maximum(by1c, by1r), 0.0)
    inter = wx * wy
    iou = inter / (area_c + area_r - inter + 1e-9)
    act_pair = (iota_p < f32(PRE)) & (iota_e1 < f32(PRE))
    sup_u = ((iou > NMS_THRESH) & (iota_p < iota_e1) & act_pair).astype(bf16)

    iota_row = lax.broadcasted_iota(jnp.int32, (1, CAP), 1).astype(f32)
    act_row = (iota_row < f32(PRE)).astype(f32)

    def nms_cond(carry):
        _, done = carry
        return jnp.logical_not(done)

    def nms_body(carry):
        keep, _ = carry
        cnt = lax.dot_general(keep.astype(bf16), sup_u,
                              (((1,), (0,)), ((), ())),
                              preferred_element_type=f32)
        newk = jnp.where(cnt == 0.0, act_row, 0.0)
        done = jnp.all(newk == keep)
        return (newk, done)

    keep, _ = lax.while_loop(nms_cond, nms_body,
                             (act_row, jnp.bool_(False)))

    u_cap = (iota_p < iota_e1).astype(f32)
    pk = lax.dot_general(keep, u_cap, (((1,), (0,)), ((), ())), precision=LO)
    nk = jnp.sum(keep)
    notk = act_row * (1.0 - keep)
    pn = lax.dot_general(notk, u_cap, (((1,), (0,)), ((), ())), precision=LO)
    fpos = jnp.where(keep == 1.0, pk, nk + pn)
    fposx = jnp.where(act_row == 1.0, fpos, f32(2.0 * CAP))
    h_mat = (jnp.broadcast_to(fposx, (CAP, CAP)) == iota_p).astype(bf16)
    sc_fin = jnp.where(keep == 1.0, sc_r, f32(NEG_INF_SCORE))
    s_out = jnp.concatenate(
        [bx1r, by1r, bx2r, by2r, sc_fin,
         jnp.zeros((3, CAP), f32)], axis=0)
    out_ref[...] = onehot_matmul(s_out, h_mat)


@functools.partial(jax.jit, static_argnames=("interpret",))
def _rpn_call(sc2, ax1, ay1, ax2, ay2, dx, dy, dw, dh, interpret=False):
    return pl.pallas_call(
        _rpn_body,
        out_shape=jax.ShapeDtypeStruct((8, CAP), jnp.float32),
        interpret=interpret,
    )(sc2, ax1, ay1, ax2, ay2, dx, dy, dw, dh)


def kernel(anchors, deltas, objectness, interpret=False):
    f32 = jnp.float32
    sc = jnp.full((NPAD,), -jnp.inf, f32).at[:N_ANCHORS].set(objectness)
    sc2 = sc.reshape(ROWS, LANES)

    def cols(m):
        mp = jnp.zeros((NPAD, 4), f32).at[:N_ANCHORS].set(m)
        return [mp[:, j].reshape(ROWS, LANES) for j in range(4)]

    ax1, ay1, ax2, ay2 = cols(anchors)
    dx, dy, dw, dh = cols(deltas)
    res = _rpn_call(sc2, ax1, ay1, ax2, ay2, dx, dy, dw, dh,
                    interpret=interpret)
    return jnp.transpose(res[0:5, :PRE])


# speedup vs baseline: 39.7086x; 1.0380x over previous
import functools

import jax
import jax.numpy as jnp
import numpy as np
from jax import lax
from jax.experimental import pallas as pl

N_ANCHORS = 20000
NPAD = 20480
ROWS = 160
LANES = 128
PRE = 1000
CAP = 1024
NMS_THRESH = 0.7
IMG_H = 800.0
IMG_W = 800.0
BBOX_XFORM_CLIP = float(np.log(1000.0 / 16.0))
NEG_INF_SCORE = -1e9
HI = jax.lax.Precision.HIGHEST
LO = jax.lax.Precision.DEFAULT


def _rpn_body(sc_ref, ax1_ref, ay1_ref, ax2_ref, ay2_ref,
              dx_ref, dy_ref, dw_ref, dh_ref, out_ref):
    f32 = jnp.float32
    score = sc_ref[...]
    ax1 = ax1_ref[...]
    ay1 = ay1_ref[...]
    ax2 = ax2_ref[...]
    ay2 = ay2_ref[...]
    dx = dx_ref[...]
    dy = dy_ref[...]
    dw = dw_ref[...]
    dh = dh_ref[...]

    w = ax2 - ax1
    h = ay2 - ay1
    cx = ax1 + 0.5 * w
    cy = ay1 + 0.5 * h
    dwc = jnp.minimum(dw, BBOX_XFORM_CLIP)
    dhc = jnp.minimum(dh, BBOX_XFORM_CLIP)
    pcx = dx * w + cx
    pcy = dy * h + cy
    pw = jnp.exp(dwc) * w
    ph = jnp.exp(dhc) * h
    x1 = jnp.clip(pcx - 0.5 * pw, 0.0, IMG_W)
    y1 = jnp.clip(pcy - 0.5 * ph, 0.0, IMG_H)
    x2 = jnp.clip(pcx + 0.5 * pw, 0.0, IMG_W)
    y2 = jnp.clip(pcy + 0.5 * ph, 0.0, IMG_H)

    int_min = jnp.int32(-2147483648)
    kb = lax.bitcast_convert_type(score, jnp.int32)
    k = jnp.where(kb >= 0, kb, kb ^ jnp.int32(0x7FFFFFFF))

    def bs_body(b, u):
        bit = lax.shift_left(jnp.int32(1), jnp.int32(31) - b)
        uc = u | bit
        cand = uc ^ int_min
        cnt = jnp.sum((k >= cand).astype(jnp.int32))
        return jnp.where(cnt >= PRE, uc, u)

    u_fin = lax.fori_loop(0, 32, bs_body, jnp.int32(0))
    thr = u_fin ^ int_min
    sel = k >= thr
    self_f = sel.astype(f32)

    io0_l = lax.broadcasted_iota(jnp.int32, (LANES, LANES), 0)
    io1_l = lax.broadcasted_iota(jnp.int32, (LANES, LANES), 1)
    u128 = (io0_l < io1_l).astype(f32)
    within = lax.dot_general(self_f, u128, (((1,), (0,)), ((), ())),
                             precision=LO)
    rowtot = jnp.sum(self_f, axis=1, keepdims=True)
    io0_r = lax.broadcasted_iota(jnp.int32, (ROWS, ROWS), 0)
    io1_r = lax.broadcasted_iota(jnp.int32, (ROWS, ROWS), 1)
    l160 = (io1_r < io0_r).astype(f32)
    rowoff = lax.dot_general(l160, rowtot, (((1,), (0,)), ((), ())),
                             precision=LO)
    pos = within + rowoff
    posx = jnp.where(sel, pos, f32(2.0 * CAP))

    idxf = (lax.broadcasted_iota(jnp.int32, (ROWS, LANES), 0) * LANES
            + lax.broadcasted_iota(jnp.int32, (ROWS, LANES), 1)).astype(f32)

    iota_p = lax.broadcasted_iota(jnp.int32, (CAP, CAP), 0).astype(f32)
    iota_e1 = lax.broadcasted_iota(jnp.int32, (CAP, CAP), 1).astype(f32)
    bf16 = jnp.bfloat16

    def split3(x):
        hi = x.astype(bf16)
        r1 = x - hi.astype(f32)
        mid = r1.astype(bf16)
        lo = (r1 - mid.astype(f32)).astype(bf16)
        return jnp.concatenate([hi, mid, lo], axis=0)

    def onehot_matmul(vals, onehot_bf):
        m = vals.shape[0]
        parts = lax.dot_general(split3(vals), onehot_bf,
                                (((1,), (1,)), ((), ())),
                                preferred_element_type=f32)
        return parts[0:m] + parts[m:2 * m] + parts[2 * m:3 * m]

    score_pay = jnp.where(sel, score, 0.0)
    payload = (x1, y1, x2, y2, score_pay, idxf,
               jnp.ones((ROWS, LANES), f32), jnp.zeros((ROWS, LANES), f32))

    def row1024(arr, b):
        ch = arr[8 * b:8 * b + 8, :]
        return jnp.concatenate([ch[s:s + 1, :] for s in range(8)], axis=1)

    c_mat = jnp.zeros((8, CAP), f32)
    for b in range(NPAD // CAP):
        pr = row1024(posx, b)
        ob = (jnp.broadcast_to(pr, (CAP, CAP)) == iota_p).astype(bf16)
        vb = jnp.concatenate([row1024(q, b) for q in payload], axis=0)
        c_mat = c_mat + onehot_matmul(vb, ob)

    eye = (iota_p == iota_e1).astype(f32)

    def to_col(row):
        return jnp.sum(jnp.broadcast_to(row, (CAP, CAP)) * eye,
                       axis=1, keepdims=True)

    s_row = c_mat[4:5, :]
    i_row = c_mat[5:6, :]
    v_row = c_mat[6:7, :]
    s_col = to_col(s_row)
    i_col = to_col(i_row)
    v_col = to_col(v_row)
    before = ((v_col > v_row)
              | ((v_col == v_row)
                 & ((s_col > s_row)
                    | ((s_col == s_row)
                       & ((i_col < i_row)
                          | ((i_col == i_row) & (iota_p < iota_e1)))))))
    rank_row = jnp.sum(before.astype(f32), axis=0, keepdims=True)
    g_mat = (jnp.broadcast_to(rank_row, (CAP, CAP)) == iota_p).astype(bf16)
    s_sorted = onehot_matmul(c_mat, g_mat)

    bx1r = s_sorted[0:1, :]
    by1r = s_sorted[1:2, :]
    bx2r = s_sorted[2:3, :]
    by2r = s_sorted[3:4, :]
    sc_r = s_sorted[4:5, :]
    bx1c = to_col(bx1r)
    by1c = to_col(by1r)
    bx2c = to_col(bx2r)
    by2c = to_col(by2r)
    area_r = (bx2r - bx1r) * (by2r - by1r)
    area_c = (bx2c - bx1c) * (by2c - by1c)
    wx = jnp.maximum(jnp.minimum(bx2c, bx2r) - jnp.maximum(bx1c, bx1r), 0.0)
    wy = jnp.maximum(jnp.minimum(by2c, by2r) - jnp.maximum(by1c, by1r), 0.0)
    inter = wx * wy
    iou = inter / (area_c + area_r - inter + 1e-9)
    act_pair = (iota_p < f32(PRE)) & (iota_e1 < f32(PRE))
    sup_u = ((iou > NMS_THRESH) & (iota_p < iota_e1) & act_pair).astype(bf16)

    iota_row = lax.broadcasted_iota(jnp.int32, (1, CAP), 1).astype(f32)
    act_row = (iota_row < f32(PRE)).astype(f32)

    def nms_cond(carry):
        _, done = carry
        return jnp.logical_not(done)

    def nms_step(keep):
        cnt = lax.dot_general(keep.astype(bf16), sup_u,
                              (((1,), (0,)), ((), ())),
                              preferred_element_type=f32)
        return jnp.where(cnt == 0.0, act_row, 0.0)

    def nms_body(carry):
        keep, _ = carry
        k1 = nms_step(keep)
        k2 = nms_step(k1)
        k3 = nms_step(k2)
        done = jnp.all(k3 == k2)
        return (k3, done)

    keep, _ = lax.while_loop(nms_cond, nms_body,
                             (act_row, jnp.bool_(False)))

    u_cap = (iota_p < iota_e1).astype(f32)
    pk = lax.dot_general(keep, u_cap, (((1,), (0,)), ((), ())), precision=LO)
    nk = jnp.sum(keep)
    notk = act_row * (1.0 - keep)
    pn = lax.dot_general(notk, u_cap, (((1,), (0,)), ((), ())), precision=LO)
    fpos = jnp.where(keep == 1.0, pk, nk + pn)
    fposx = jnp.where(act_row == 1.0, fpos, f32(2.0 * CAP))
    h_mat = (jnp.broadcast_to(fposx, (CAP, CAP)) == iota_p).astype(bf16)
    sc_fin = jnp.where(keep == 1.0, sc_r, f32(NEG_INF_SCORE))
    s_out = jnp.concatenate(
        [bx1r, by1r, bx2r, by2r, sc_fin,
         jnp.zeros((3, CAP), f32)], axis=0)
    out_ref[...] = onehot_matmul(s_out, h_mat)


@functools.partial(jax.jit, static_argnames=("interpret",))
def _rpn_call(sc2, ax1, ay1, ax2, ay2, dx, dy, dw, dh, interpret=False):
    return pl.pallas_call(
        _rpn_body,
        out_shape=jax.ShapeDtypeStruct((8, CAP), jnp.float32),
        interpret=interpret,
    )(sc2, ax1, ay1, ax2, ay2, dx, dy, dw, dh)


def kernel(anchors, deltas, objectness, interpret=False):
    f32 = jnp.float32
    sc = jnp.full((NPAD,), -jnp.inf, f32).at[:N_ANCHORS].set(objectness)
    sc2 = sc.reshape(ROWS, LANES)

    def cols(m):
        mp = jnp.zeros((NPAD, 4), f32).at[:N_ANCHORS].set(m)
        return [mp[:, j].reshape(ROWS, LANES) for j in range(4)]

    ax1, ay1, ax2, ay2 = cols(anchors)
    dx, dy, dw, dh = cols(deltas)
    res = _rpn_call(sc2, ax1, ay1, ax2, ay2, dx, dy, dw, dh,
                    interpret=interpret)
    return jnp.transpose(res[0:5, :PRE])


# speedup vs baseline: 39.7594x; 1.0013x over previous
import jax
import jax.numpy as jnp
import numpy as np
from jax import lax
from jax.experimental import pallas as pl

N_ANCHORS = 20000
NPAD = 20480
ROWS = 160
LANES = 128
PRE = 1000
CAP = 1024
NMS_THRESH = 0.7
IMG_H = 800.0
IMG_W = 800.0
BBOX_XFORM_CLIP = float(np.log(1000.0 / 16.0))
NEG_INF_SCORE = -1e9
HI = jax.lax.Precision.HIGHEST
LO = jax.lax.Precision.DEFAULT


def _rpn_body(sc_ref, ax1_ref, ay1_ref, ax2_ref, ay2_ref,
              dx_ref, dy_ref, dw_ref, dh_ref, out_ref):
    f32 = jnp.float32
    score = sc_ref[...]
    ax1 = ax1_ref[...]
    ay1 = ay1_ref[...]
    ax2 = ax2_ref[...]
    ay2 = ay2_ref[...]
    dx = dx_ref[...]
    dy = dy_ref[...]
    dw = dw_ref[...]
    dh = dh_ref[...]

    w = ax2 - ax1
    h = ay2 - ay1
    cx = ax1 + 0.5 * w
    cy = ay1 + 0.5 * h
    dwc = jnp.minimum(dw, BBOX_XFORM_CLIP)
    dhc = jnp.minimum(dh, BBOX_XFORM_CLIP)
    pcx = dx * w + cx
    pcy = dy * h + cy
    pw = jnp.exp(dwc) * w
    ph = jnp.exp(dhc) * h
    x1 = jnp.clip(pcx - 0.5 * pw, 0.0, IMG_W)
    y1 = jnp.clip(pcy - 0.5 * ph, 0.0, IMG_H)
    x2 = jnp.clip(pcx + 0.5 * pw, 0.0, IMG_W)
    y2 = jnp.clip(pcy + 0.5 * ph, 0.0, IMG_H)

    int_min = jnp.int32(-2147483648)
    kb = lax.bitcast_convert_type(score, jnp.int32)
    k = jnp.where(kb >= 0, kb, kb ^ jnp.int32(0x7FFFFFFF))

    def bs_body(b, u):
        bit = lax.shift_left(jnp.int32(1), jnp.int32(31) - b)
        uc = u | bit
        cand = uc ^ int_min
        cnt = jnp.sum((k >= cand).astype(jnp.int32))
        return jnp.where(cnt >= PRE, uc, u)

    u_fin = lax.fori_loop(0, 32, bs_body, jnp.int32(0))
    thr = u_fin ^ int_min
    sel = k >= thr
    self_f = sel.astype(f32)

    io0_l = lax.broadcasted_iota(jnp.int32, (LANES, LANES), 0)
    io1_l = lax.broadcasted_iota(jnp.int32, (LANES, LANES), 1)
    u128 = (io0_l < io1_l).astype(f32)
    within = lax.dot_general(self_f, u128, (((1,), (0,)), ((), ())),
                             precision=LO)
    rowtot = jnp.sum(self_f, axis=1, keepdims=True)
    io0_r = lax.broadcasted_iota(jnp.int32, (ROWS, ROWS), 0)
    io1_r = lax.broadcasted_iota(jnp.int32, (ROWS, ROWS), 1)
    l160 = (io1_r < io0_r).astype(f32)
    rowoff = lax.dot_general(l160, rowtot, (((1,), (0,)), ((), ())),
                             precision=LO)
    pos = within + rowoff
    posx = jnp.where(sel, pos, f32(2.0 * CAP))

    idxf = (lax.broadcasted_iota(jnp.int32, (ROWS, LANES), 0) * LANES
            + lax.broadcasted_iota(jnp.int32, (ROWS, LANES), 1)).astype(f32)

    iota_p = lax.broadcasted_iota(jnp.int32, (CAP, CAP), 0).astype(f32)
    iota_e1 = lax.broadcasted_iota(jnp.int32, (CAP, CAP), 1).astype(f32)
    bf16 = jnp.bfloat16

    def split3(x):
        hi = x.astype(bf16)
        r1 = x - hi.astype(f32)
        mid = r1.astype(bf16)
        lo = (r1 - mid.astype(f32)).astype(bf16)
        return jnp.concatenate([hi, mid, lo], axis=0)

    def onehot_matmul(vals, onehot_bf):
        m = vals.shape[0]
        parts = lax.dot_general(split3(vals), onehot_bf,
                                (((1,), (1,)), ((), ())),
                                preferred_element_type=f32)
        return parts[0:m] + parts[m:2 * m] + parts[2 * m:3 * m]

    score_pay = jnp.where(sel, score, 0.0)
    payload = (x1, y1, x2, y2, score_pay, idxf,
               jnp.ones((ROWS, LANES), f32), jnp.zeros((ROWS, LANES), f32))

    def row1024(arr, b):
        ch = arr[8 * b:8 * b + 8, :]
        return jnp.concatenate([ch[s:s + 1, :] for s in range(8)], axis=1)

    c_mat = jnp.zeros((8, CAP), f32)
    for b in range(NPAD // CAP):
        pr = row1024(posx, b)
        ob = (jnp.broadcast_to(pr, (CAP, CAP)) == iota_p).astype(bf16)
        vb = jnp.concatenate([row1024(q, b) for q in payload], axis=0)
        c_mat = c_mat + onehot_matmul(vb, ob)

    eye = (iota_p == iota_e1).astype(f32)

    def to_col(row):
        return jnp.sum(jnp.broadcast_to(row, (CAP, CAP)) * eye,
                       axis=1, keepdims=True)

    s_row = c_mat[4:5, :]
    i_row = c_mat[5:6, :]
    v_row = c_mat[6:7, :]
    s_col = to_col(s_row)
    i_col = to_col(i_row)
    v_col = to_col(v_row)
    before = ((v_col > v_row)
              | ((v_col == v_row)
                 & ((s_col > s_row)
                    | ((s_col == s_row)
                       & ((i_col < i_row)
                          | ((i_col == i_row) & (iota_p < iota_e1)))))))
    rank_row = jnp.sum(before.astype(f32), axis=0, keepdims=True)
    g_mat = (jnp.broadcast_to(rank_row, (CAP, CAP)) == iota_p).astype(bf16)
    s_sorted = onehot_matmul(c_mat, g_mat)

    bx1r = s_sorted[0:1, :]
    by1r = s_sorted[1:2, :]
    bx2r = s_sorted[2:3, :]
    by2r = s_sorted[3:4, :]
    sc_r = s_sorted[4:5, :]
    bx1c = to_col(bx1r)
    by1c = to_col(by1r)
    bx2c = to_col(bx2r)
    by2c = to_col(by2r)
    area_r = (bx2r - bx1r) * (by2r - by1r)
    area_c = (bx2c - bx1c) * (by2c - by1c)
    wx = jnp.maximum(jnp.minimum(bx2c, bx2r) - jnp.maximum(bx1c, bx1r), 0.0)
    wy = jnp.maximum(jnp.minimum(by2c, by2r) - jnp.maximum(by1c, by1r), 0.0)
    inter = wx * wy
    iou = inter / (area_c + area_r - inter + 1e-9)
    act_pair = (iota_p < f32(PRE)) & (iota_e1 < f32(PRE))
    sup_u = ((iou > NMS_THRESH) & (iota_p < iota_e1) & act_pair).astype(bf16)

    iota_row = lax.broadcasted_iota(jnp.int32, (1, CAP), 1).astype(f32)
    act_row = (iota_row < f32(PRE)).astype(f32)

    def nms_cond(carry):
        _, done = carry
        return jnp.logical_not(done)

    def nms_step(keep):
        cnt = lax.dot_general(keep.astype(bf16), sup_u,
                              (((1,), (0,)), ((), ())),
                              preferred_element_type=f32)
        return jnp.where(cnt == 0.0, act_row, 0.0)

    def nms_body(carry):
        keep, _ = carry
        k1 = nms_step(keep)
        k2 = nms_step(k1)
        k3 = nms_step(k2)
        done = jnp.all(k3 == k2)
        return (k3, done)

    keep, _ = lax.while_loop(nms_cond, nms_body,
                             (act_row, jnp.bool_(False)))

    u_cap = (iota_p < iota_e1).astype(f32)
    pk = lax.dot_general(keep, u_cap, (((1,), (0,)), ((), ())), precision=LO)
    nk = jnp.sum(keep)
    notk = act_row * (1.0 - keep)
    pn = lax.dot_general(notk, u_cap, (((1,), (0,)), ((), ())), precision=LO)
    fpos = jnp.where(keep == 1.0, pk, nk + pn)
    fposx = jnp.where(act_row == 1.0, fpos, f32(2.0 * CAP))
    h_mat = (jnp.broadcast_to(fposx, (CAP, CAP)) == iota_p).astype(bf16)
    sc_fin = jnp.where(keep == 1.0, sc_r, f32(NEG_INF_SCORE))
    s_out = jnp.concatenate(
        [bx1r, by1r, bx2r, by2r, sc_fin,
         jnp.zeros((3, CAP), f32)], axis=0)
    out_ref[...] = onehot_matmul(s_out, h_mat)


@jax.jit
def _rpn_call(sc2, ax1, ay1, ax2, ay2, dx, dy, dw, dh):
    return pl.pallas_call(
        _rpn_body,
        out_shape=jax.ShapeDtypeStruct((8, CAP), jnp.float32),
    )(sc2, ax1, ay1, ax2, ay2, dx, dy, dw, dh)


def kernel(anchors, deltas, objectness):
    f32 = jnp.float32
    sc = jnp.full((NPAD,), -jnp.inf, f32).at[:N_ANCHORS].set(objectness)
    sc2 = sc.reshape(ROWS, LANES)

    def cols(m):
        mp = jnp.zeros((NPAD, 4), f32).at[:N_ANCHORS].set(m)
        return [mp[:, j].reshape(ROWS, LANES) for j in range(4)]

    ax1, ay1, ax2, ay2 = cols(anchors)
    dx, dy, dw, dh = cols(deltas)
    res = _rpn_call(sc2, ax1, ay1, ax2, ay2, dx, dy, dw, dh)
    return jnp.transpose(res[0:5, :PRE])
